# Initial kernel scaffold; baseline (speedup 1.0000x reference)
import jax
import jax.numpy as jnp
from jax.experimental import pallas as pl


def kernel(x, params):
    raise NotImplementedError("write your pallas kernel here")


# speedup vs baseline: 3.4930x; 3.4930x over previous
import jax
import jax.numpy as jnp
from jax.experimental import pallas as pl
from jax.experimental.pallas import tpu as pltpu
from jax.experimental.pallas import tpu_sc as plsc

N = 10000
K = 32
PROP = 32
SPACE = 4

MLP_RB = 2000
KNN_RB = 200
GATHER_W = 128

_BIG = 3.0e38


def _rows(rb, cols):
    return pl.BlockSpec((rb, cols), lambda i: (i, 0))


def _full(r, c):
    return pl.BlockSpec((r, c), lambda i: (0, 0))


def _front_body(x_ref, w1, b1, w2, b2, o_ref):
    h = jnp.maximum(
        jnp.dot(x_ref[...], w1[...], preferred_element_type=jnp.float32) + b1[...], 0.0)
    o_ref[...] = jnp.maximum(
        jnp.dot(h, w2[...], preferred_element_type=jnp.float32) + b2[...], 0.0)


def _front(x, w1, b1, w2, b2):
    d1, d2 = w1.shape[0], w2.shape[1]
    return pl.pallas_call(
        _front_body,
        grid=(N // MLP_RB,),
        in_specs=[_rows(MLP_RB, d1), _full(d1, w1.shape[1]), _full(1, w1.shape[1]),
                  _full(w2.shape[0], d2), _full(1, d2)],
        out_specs=_rows(MLP_RB, d2),
        out_shape=jax.ShapeDtypeStruct((N, d2), jnp.float32),
    )(x, w1, b1.reshape(1, -1), w2, b2.reshape(1, -1))


def _proj_body(x_ref, ws, bs, wh, bh, q_ref, c_ref, h32_ref):
    x = x_ref[...]
    s = jnp.dot(x, ws[...], preferred_element_type=jnp.float32) + bs[...]
    h32_ref[...] = jnp.dot(x, wh[...], preferred_element_type=jnp.float32) + bh[...]
    sq = jnp.sum(s * s, axis=1, keepdims=True)
    one = jnp.ones_like(sq)
    zero = jnp.zeros_like(sq)
    q_ref[...] = jnp.concatenate([-2.0 * s, one, sq, zero, zero], axis=1)
    c_ref[...] = jnp.concatenate([s, sq, one, zero, zero], axis=1)


def _proj(x, ws, bs, wh, bh):
    gin = x.shape[1]
    return pl.pallas_call(
        _proj_body,
        grid=(N // MLP_RB,),
        in_specs=[_rows(MLP_RB, gin), _full(gin, SPACE), _full(1, SPACE),
                  _full(gin, PROP), _full(1, PROP)],
        out_specs=[_rows(MLP_RB, 8), _rows(MLP_RB, 8), _rows(MLP_RB, PROP)],
        out_shape=[jax.ShapeDtypeStruct((N, 8), jnp.float32),
                   jax.ShapeDtypeStruct((N, 8), jnp.float32),
                   jax.ShapeDtypeStruct((N, PROP), jnp.float32)],
    )(x, ws, bs.reshape(1, -1), wh, bh.reshape(1, -1))


def _knn_body(qb_ref, c_ref, idx_ref, w_ref, d_ref):
    d_ref[...] = jax.lax.dot_general(
        qb_ref[...], c_ref[...], (((1,), (1,)), ((), ())),
        preferred_element_type=jnp.float32)
    col = jax.lax.broadcasted_iota(jnp.int32, (KNN_RB, N), 1)
    for k in range(K):
        d = d_ref[...]
        m = jnp.min(d, axis=1, keepdims=True)
        idxv = jnp.min(jnp.where(d == m, col, N), axis=1, keepdims=True)
        idx_ref[:, k:k + 1] = idxv
        w_ref[:, k:k + 1] = jnp.exp(-10.0 * m)
        d_ref[...] = jnp.where(col == idxv, _BIG, d)


def _knn(q, c):
    return pl.pallas_call(
        _knn_body,
        grid=(N // KNN_RB,),
        in_specs=[_rows(KNN_RB, 8), _full(N, 8)],
        out_specs=[_rows(KNN_RB, K), _rows(KNN_RB, K)],
        out_shape=[jax.ShapeDtypeStruct((N, K), jnp.int32),
                   jax.ShapeDtypeStruct((N, K), jnp.float32)],
        scratch_shapes=[pltpu.VMEM((KNN_RB, N), jnp.float32)],
    )(q, c)


def _sc_gather(h32, idx):
    table = h32.reshape(N, 1, PROP)
    idx_flat = idx.reshape(1, N * K)
    mesh = plsc.VectorSubcoreMesh(core_axis_name="core", subcore_axis_name="subcore")

    @pl.kernel(out_type=jax.ShapeDtypeStruct((N * K, 1, PROP), jnp.float32), mesh=mesh)
    def kern(x_hbm, i_hbm, o_hbm):
        def body(i_vmem, o_vmem):
            pltpu.sync_copy(x_hbm.at[i_vmem.at[0]], o_vmem)

        pltpu.emit_pipeline(
            body,
            grid=(N * K // GATHER_W,),
            in_specs=[pl.BlockSpec((1, GATHER_W), index_map=lambda i: (0, i))],
            out_specs=[pl.BlockSpec((GATHER_W, 1, PROP), index_map=lambda i: (i, 0, 0))],
            core_axis_name=("core", "subcore"),
            dimension_semantics=(pltpu.PARALLEL,),
        )(i_hbm, o_hbm)

    return kern(table, idx_flat)


def _agg_body(msg_ref, w_ref, x_ref, wo_m, wo_x, wo_h, bo,
              w1, b1, w2, b2, w3, b3, o_ref):
    w = w_ref[...]
    acc = None
    mx = None
    for k in range(K):
        hk = msg_ref[:, PROP * k:PROP * (k + 1)] * w[:, k:k + 1]
        acc = hk if acc is None else acc + hk
        mx = hk if mx is None else jnp.maximum(mx, hk)
    mean = acc * (1.0 / K)
    y = (jnp.dot(mean, wo_m[...], preferred_element_type=jnp.float32)
         + jnp.dot(mx, wo_x[...], preferred_element_type=jnp.float32)
         + jnp.dot(x_ref[...], wo_h[...], preferred_element_type=jnp.float32)
         + bo[...])
    h1 = jnp.maximum(jnp.dot(y, w1[...], preferred_element_type=jnp.float32) + b1[...], 0.0)
    h2 = jnp.maximum(jnp.dot(h1, w2[...], preferred_element_type=jnp.float32) + b2[...], 0.0)
    o_ref[...] = jnp.maximum(jnp.dot(h2, w3[...], preferred_element_type=jnp.float32) + b3[...], 0.0)


def _agg(msg, w, x, wo, bo, w1, b1, w2, b2, w3, b3):
    gin = x.shape[1]
    wo_m, wo_x, wo_h = wo[:PROP], wo[PROP:2 * PROP], wo[2 * PROP:]
    hid = w1.shape[1]
    dout = w3.shape[1]
    return pl.pallas_call(
        _agg_body,
        grid=(N // MLP_RB,),
        in_specs=[_rows(MLP_RB, K * PROP), _rows(MLP_RB, K), _rows(MLP_RB, gin),
                  _full(PROP, 2 * PROP), _full(PROP, 2 * PROP), _full(gin, 2 * PROP),
                  _full(1, 2 * PROP),
                  _full(2 * PROP, hid), _full(1, hid),
                  _full(hid, hid), _full(1, hid),
                  _full(hid, dout), _full(1, dout)],
        out_specs=_rows(MLP_RB, dout),
        out_shape=jax.ShapeDtypeStruct((N, dout), jnp.float32),
    )(msg, w, x, wo_m, wo_x, wo_h, bo.reshape(1, -1),
      w1, b1.reshape(1, -1), w2, b2.reshape(1, -1), w3, b3.reshape(1, -1))


def _final_body(x_ref, w3, b3, w4, b4, o_ref):
    h = jnp.maximum(
        jnp.dot(x_ref[...], w3[...], preferred_element_type=jnp.float32) + b3[...], 0.0)
    o_ref[...] = jnp.dot(h, w4[...], preferred_element_type=jnp.float32) + b4[...]


def _final(x, w3, b3, w4, b4):
    gin = x.shape[1]
    hid = w3.shape[1]
    dout = w4.shape[1]
    return pl.pallas_call(
        _final_body,
        grid=(N // MLP_RB,),
        in_specs=[_rows(MLP_RB, gin), _full(gin, hid), _full(1, hid),
                  _full(hid, dout), _full(1, dout)],
        out_specs=_rows(MLP_RB, dout),
        out_shape=jax.ShapeDtypeStruct((N, dout), jnp.float32),
    )(x, w3, b3.reshape(1, -1), w4, b4.reshape(1, -1))


def kernel(x, params):
    p = params
    h = _front(x, p['fc1_W'], p['fc1_b'], p['fc2_W'], p['fc2_b'])
    for i in range(4):
        g = p['gn%d' % i]
        d = p['dn%d' % i]
        q, c, h32 = _proj(h, g['Ws'], g['bs'], g['Wh'], g['bh'])
        idx, w = _knn(q, c)
        msg = _sc_gather(h32, idx).reshape(N, K * PROP)
        h = _agg(msg, w, h, g['Wo'], g['bo'],
                 d['W1'], d['b1'], d['W2'], d['b2'], d['W3'], d['b3'])
    return _final(h, p['fc3_W'], p['fc3_b'], p['fc4_W'], p['fc4_b'])
